# baseline (speedup 1.0000x reference)
import jax
import jax.numpy as jnp
from jax.experimental import pallas as pl

_B, _V = 8, 1000000
_W = 500
_NSTEP = _V // (8 * _W)
_EPS = 1e-10


def _rotl(x, d):
    return (x << jnp.uint32(d)) | (x >> jnp.uint32(32 - d))


def _threefry_bits(idx):
    ks0 = jnp.uint32(0)
    ks1 = jnp.uint32(1)
    ks2 = jnp.uint32(0x1BD11BDA) ^ ks0 ^ ks1
    ks = (ks0, ks1, ks2)
    rots = ((13, 15, 26, 6), (17, 29, 16, 24))
    x0 = jnp.zeros_like(idx) + ks0
    x1 = idx + ks1
    for i in range(5):
        for r in rots[i % 2]:
            x0 = x0 + x1
            x1 = _rotl(x1, r) ^ x0
        x0 = x0 + ks[(i + 1) % 3]
        x1 = x1 + ks[(i + 2) % 3] + jnp.uint32(i + 1)
    return x0 ^ x1


def _row_kernel(a_ref, y_ref, ss_ref):
    row = pl.program_id(0)
    eps = jnp.float32(_EPS)

    rw = jax.lax.broadcasted_iota(jnp.uint32, (8, _W), 0) * jnp.uint32(_W) + \
         jax.lax.broadcasted_iota(jnp.uint32, (8, _W), 1)
    row_base = (row * _V).astype(jnp.uint32)

    def p1(k, carry):
        mg, ma = carry
        a = a_ref[0, k]
        idx = rw + (row_base + k.astype(jnp.uint32) * jnp.uint32(8 * _W))
        bits = _threefry_bits(idx)
        u = jax.lax.bitcast_convert_type(
            (bits >> jnp.uint32(9)) | jnp.uint32(0x3F800000), jnp.float32
        ) - jnp.float32(1.0)
        g = a - jnp.log(eps - jnp.log(u + eps))
        y_ref[0, k] = g
        return jnp.maximum(mg, jnp.max(g)), jnp.maximum(ma, jnp.max(a))

    mg, ma = jax.lax.fori_loop(
        0, _NSTEP, p1, (jnp.float32(-jnp.inf), jnp.float32(-jnp.inf)))

    def p2(k, carry):
        sg, sa = carry
        e = jnp.exp(y_ref[0, k] - mg)
        y_ref[0, k] = e
        e2 = jnp.exp(a_ref[0, k] - ma)
        ss_ref[0, k] = e2
        return sg + jnp.sum(e), sa + jnp.sum(e2)

    sg, sa = jax.lax.fori_loop(0, _NSTEP, p2, (jnp.float32(0.0), jnp.float32(0.0)))

    rg = jnp.float32(1.0) / sg
    ra = jnp.float32(1.0) / sa

    def p3(k, _):
        y_ref[0, k] = jnp.maximum(y_ref[0, k] * rg, eps)
        ss_ref[0, k] = ss_ref[0, k] * ra
        return 0

    jax.lax.fori_loop(0, _NSTEP, p3, 0)


def kernel(alpha):
    a4 = alpha.reshape(_B, _NSTEP, 8, _W)
    y, ss = pl.pallas_call(
        _row_kernel,
        grid=(_B,),
        in_specs=[pl.BlockSpec((1, _NSTEP, 8, _W), lambda i: (i, 0, 0, 0))],
        out_specs=[
            pl.BlockSpec((1, _NSTEP, 8, _W), lambda i: (i, 0, 0, 0)),
            pl.BlockSpec((1, _NSTEP, 8, _W), lambda i: (i, 0, 0, 0)),
        ],
        out_shape=[
            jax.ShapeDtypeStruct((_B, _NSTEP, 8, _W), jnp.float32),
            jax.ShapeDtypeStruct((_B, _NSTEP, 8, _W), jnp.float32),
        ],
    )(a4)
    y = y.reshape(_B, _V)
    ss = ss.reshape(_B, _V)
    return (y, y, ss)


# speedup vs baseline: 1.2104x; 1.2104x over previous
import jax
import jax.numpy as jnp
from jax.experimental import pallas as pl

_B, _V = 8, 1000000
_W = 1000
_NSTEP = _V // (8 * _W)
_EPS = 1e-10


def _rotl(x, d):
    return (x << jnp.uint32(d)) | (x >> jnp.uint32(32 - d))


def _threefry_bits(idx):
    ks0 = jnp.uint32(0)
    ks1 = jnp.uint32(1)
    ks2 = jnp.uint32(0x1BD11BDA) ^ ks0 ^ ks1
    ks = (ks0, ks1, ks2)
    rots = ((13, 15, 26, 6), (17, 29, 16, 24))
    x0 = jnp.zeros_like(idx) + ks0
    x1 = idx + ks1
    for i in range(5):
        for r in rots[i % 2]:
            x0 = x0 + x1
            x1 = _rotl(x1, r) ^ x0
        x0 = x0 + ks[(i + 1) % 3]
        x1 = x1 + ks[(i + 2) % 3] + jnp.uint32(i + 1)
    return x0 ^ x1


def _row_kernel(a_ref, y_ref, ss_ref):
    row = pl.program_id(0)
    eps = jnp.float32(_EPS)

    rw = jax.lax.broadcasted_iota(jnp.uint32, (8, _W), 0) * jnp.uint32(_W) + \
         jax.lax.broadcasted_iota(jnp.uint32, (8, _W), 1)
    row_base = (row * _V).astype(jnp.uint32)

    ninf = jnp.full((8, _W), -jnp.inf, jnp.float32)

    def p1(k, carry):
        mg, ma = carry
        a = a_ref[0, k]
        idx = rw + (row_base + k.astype(jnp.uint32) * jnp.uint32(8 * _W))
        bits = _threefry_bits(idx)
        u = jax.lax.bitcast_convert_type(
            (bits >> jnp.uint32(9)) | jnp.uint32(0x3F800000), jnp.float32
        ) - jnp.float32(1.0)
        g = a - jnp.log(eps - jnp.log(u + eps))
        y_ref[0, k] = g
        return jnp.maximum(mg, g), jnp.maximum(ma, a)

    mg_acc, ma_acc = jax.lax.fori_loop(0, _NSTEP, p1, (ninf, ninf))
    mg = jnp.max(mg_acc)
    ma = jnp.max(ma_acc)

    zero = jnp.zeros((8, _W), jnp.float32)

    def p2(k, carry):
        sg, sa = carry
        e = jnp.exp(y_ref[0, k] - mg)
        y_ref[0, k] = e
        e2 = jnp.exp(a_ref[0, k] - ma)
        ss_ref[0, k] = e2
        return sg + e, sa + e2

    sg_acc, sa_acc = jax.lax.fori_loop(0, _NSTEP, p2, (zero, zero))
    sg = jnp.sum(sg_acc)
    sa = jnp.sum(sa_acc)

    rg = jnp.float32(1.0) / sg
    ra = jnp.float32(1.0) / sa

    def p3(k, _):
        y_ref[0, k] = jnp.maximum(y_ref[0, k] * rg, eps)
        ss_ref[0, k] = ss_ref[0, k] * ra
        return 0

    jax.lax.fori_loop(0, _NSTEP, p3, 0)


def kernel(alpha):
    a4 = alpha.reshape(_B, _NSTEP, 8, _W)
    y, ss = pl.pallas_call(
        _row_kernel,
        grid=(_B,),
        in_specs=[pl.BlockSpec((1, _NSTEP, 8, _W), lambda i: (i, 0, 0, 0))],
        out_specs=[
            pl.BlockSpec((1, _NSTEP, 8, _W), lambda i: (i, 0, 0, 0)),
            pl.BlockSpec((1, _NSTEP, 8, _W), lambda i: (i, 0, 0, 0)),
        ],
        out_shape=[
            jax.ShapeDtypeStruct((_B, _NSTEP, 8, _W), jnp.float32),
            jax.ShapeDtypeStruct((_B, _NSTEP, 8, _W), jnp.float32),
        ],
    )(a4)
    y = y.reshape(_B, _V)
    ss = ss.reshape(_B, _V)
    return (y, y, ss)


# speedup vs baseline: 12.9712x; 10.7166x over previous
import jax
import jax.numpy as jnp
from jax.experimental import pallas as pl
from jax.experimental.pallas import tpu as pltpu

_B, _V = 8, 1000000
_W = 32768
_NC = 31
_EPS = 1e-10


def _rotl(x, d):
    return (x << jnp.uint32(d)) | (x >> jnp.uint32(32 - d))


def _threefry_bits(idx):
    ks0 = jnp.uint32(0)
    ks1 = jnp.uint32(1)
    ks2 = jnp.uint32(0x1BD11BDA) ^ ks0 ^ ks1
    ks = (ks0, ks1, ks2)
    rots = ((13, 15, 26, 6), (17, 29, 16, 24))
    x0 = jnp.zeros_like(idx) + ks0
    x1 = idx + ks1
    for i in range(5):
        for r in rots[i % 2]:
            x0 = x0 + x1
            x1 = _rotl(x1, r) ^ x0
        x0 = x0 + ks[(i + 1) % 3]
        x1 = x1 + ks[(i + 2) % 3] + jnp.uint32(i + 1)
    return x0 ^ x1


def _kernel(a_ref, y_ref, ss_ref,
            g_s, mga, maa, sga, saa, mg_s, ma_s, rg_s, ra_s):
    p = pl.program_id(0)
    k = pl.program_id(1)
    eps = jnp.float32(_EPS)
    ninf = jnp.float32(-jnp.inf)

    lane = jax.lax.broadcasted_iota(jnp.int32, (8, _W), 1)
    mask = (k * _W + lane) < _V
    cols = pl.ds(k * _W, _W)

    @pl.when(p == 0)
    def _p0():
        @pl.when(k == 0)
        def _init():
            mga[...] = jnp.full((8, _W), ninf, jnp.float32)
            maa[...] = jnp.full((8, _W), ninf, jnp.float32)

        a = jnp.where(mask, a_ref[...], ninf)
        ridx = jax.lax.broadcasted_iota(jnp.uint32, (8, _W), 0) * jnp.uint32(_V)
        idx = ridx + lane.astype(jnp.uint32) + jnp.uint32(_W) * k.astype(jnp.uint32)
        bits = _threefry_bits(idx)
        u = jax.lax.bitcast_convert_type(
            (bits >> jnp.uint32(9)) | jnp.uint32(0x3F800000), jnp.float32
        ) - jnp.float32(1.0)
        g = a - jnp.log(eps - jnp.log(u + eps))
        g_s[:, cols] = g
        mga[...] = jnp.maximum(mga[...], g)
        maa[...] = jnp.maximum(maa[...], a)

    @pl.when(p == 1)
    def _p1():
        @pl.when(k == 0)
        def _stats():
            mg_s[...] = jnp.broadcast_to(
                jnp.max(mga[...], axis=1, keepdims=True), (8, 128))
            ma_s[...] = jnp.broadcast_to(
                jnp.max(maa[...], axis=1, keepdims=True), (8, 128))
            sga[...] = jnp.zeros((8, _W), jnp.float32)
            saa[...] = jnp.zeros((8, _W), jnp.float32)

        mg = mg_s[...][:, :1]
        ma = ma_s[...][:, :1]
        e = jnp.exp(g_s[:, cols] - mg)
        g_s[:, cols] = e
        sga[...] = sga[...] + e
        a = jnp.where(mask, a_ref[...], ninf)
        e2 = jnp.exp(a - ma)
        saa[...] = saa[...] + e2

    @pl.when(p == 2)
    def _p2():
        @pl.when(k == 0)
        def _recip():
            rg_s[...] = jnp.broadcast_to(
                jnp.float32(1.0)
                / jnp.sum(sga[...], axis=1, keepdims=True), (8, 128))
            ra_s[...] = jnp.broadcast_to(
                jnp.float32(1.0)
                / jnp.sum(saa[...], axis=1, keepdims=True), (8, 128))

        rg = rg_s[...][:, :1]
        ra = ra_s[...][:, :1]
        y_ref[...] = jnp.maximum(g_s[:, cols] * rg, eps)
        ma = ma_s[...][:, :1]
        a = jnp.where(mask, a_ref[...], ninf)
        ss_ref[...] = jnp.exp(a - ma) * ra


def kernel(alpha):
    y, ss = pl.pallas_call(
        _kernel,
        grid=(3, _NC),
        in_specs=[pl.BlockSpec((8, _W), lambda p, k: (0, k))],
        out_specs=[
            pl.BlockSpec((8, _W), lambda p, k: (0, (p == 2) * k)),
            pl.BlockSpec((8, _W), lambda p, k: (0, (p == 2) * k)),
        ],
        out_shape=[
            jax.ShapeDtypeStruct((_B, _V), jnp.float32),
            jax.ShapeDtypeStruct((_B, _V), jnp.float32),
        ],
        scratch_shapes=[
            pltpu.VMEM((8, _NC * _W), jnp.float32),
            pltpu.VMEM((8, _W), jnp.float32),
            pltpu.VMEM((8, _W), jnp.float32),
            pltpu.VMEM((8, _W), jnp.float32),
            pltpu.VMEM((8, _W), jnp.float32),
            pltpu.VMEM((8, 128), jnp.float32),
            pltpu.VMEM((8, 128), jnp.float32),
            pltpu.VMEM((8, 128), jnp.float32),
            pltpu.VMEM((8, 128), jnp.float32),
        ],
    )(alpha)
    return (y, y, ss)


# speedup vs baseline: 14.2054x; 1.0951x over previous
import jax
import jax.numpy as jnp
from jax.experimental import pallas as pl
from jax.experimental.pallas import tpu as pltpu

_B, _V = 8, 1000000
_W = 32768
_NC = 31
_EPS = 1e-10


def _rotl(x, d):
    return (x << jnp.uint32(d)) | (x >> jnp.uint32(32 - d))


def _threefry_bits(idx):
    ks0 = jnp.uint32(0)
    ks1 = jnp.uint32(1)
    ks2 = jnp.uint32(0x1BD11BDA) ^ ks0 ^ ks1
    ks = (ks0, ks1, ks2)
    rots = ((13, 15, 26, 6), (17, 29, 16, 24))
    x0 = jnp.zeros_like(idx) + ks0
    x1 = idx + ks1
    for i in range(5):
        for r in rots[i % 2]:
            x0 = x0 + x1
            x1 = _rotl(x1, r) ^ x0
        x0 = x0 + ks[(i + 1) % 3]
        x1 = x1 + ks[(i + 2) % 3] + jnp.uint32(i + 1)
    return x0 ^ x1


def _kernel(a_ref, y_ref, y2_ref, ss_ref,
            g_s, mga, maa, sga, saa, mg_s, ma_s, rg_s, ra_s):
    p = pl.program_id(0)
    k = pl.program_id(1)
    eps = jnp.float32(_EPS)
    ninf = jnp.float32(-jnp.inf)

    cols = pl.ds(k * _W, _W)

    @pl.when(p == 0)
    def _p0():
        @pl.when(k == 0)
        def _init():
            mga[...] = jnp.full((8, _W), ninf, jnp.float32)
            maa[...] = jnp.full((8, _W), ninf, jnp.float32)

        lane = jax.lax.broadcasted_iota(jnp.int32, (8, _W), 1)
        mask = (k * _W + lane) < _V
        a = jnp.where(mask, a_ref[...], ninf)
        ridx = jax.lax.broadcasted_iota(jnp.uint32, (8, _W), 0) * jnp.uint32(_V)
        idx = ridx + lane.astype(jnp.uint32) + jnp.uint32(_W) * k.astype(jnp.uint32)
        bits = _threefry_bits(idx)
        u = jax.lax.bitcast_convert_type(
            (bits >> jnp.uint32(9)) | jnp.uint32(0x3F800000), jnp.float32
        ) - jnp.float32(1.0)
        g = a - jnp.log(eps - jnp.log(u + eps))
        g_s[:, cols] = g
        mga[...] = jnp.maximum(mga[...], g)
        maa[...] = jnp.maximum(maa[...], a)

    @pl.when(p == 1)
    def _p1():
        @pl.when(k == 0)
        def _stats():
            mg_s[...] = jnp.broadcast_to(
                jnp.max(mga[...], axis=1, keepdims=True), (8, 128))
            ma_s[...] = jnp.broadcast_to(
                jnp.max(maa[...], axis=1, keepdims=True), (8, 128))
            sga[...] = jnp.zeros((8, _W), jnp.float32)
            saa[...] = jnp.zeros((8, _W), jnp.float32)

        mg = mg_s[...][:, :1]
        ma = ma_s[...][:, :1]
        e = jnp.exp(g_s[:, cols] - mg)
        g_s[:, cols] = e
        sga[...] = sga[...] + e
        lane = jax.lax.broadcasted_iota(jnp.int32, (8, _W), 1)
        mask = (k * _W + lane) < _V
        a = jnp.where(mask, a_ref[...], ninf)
        e2 = jnp.exp(a - ma)
        saa[...] = saa[...] + e2

    @pl.when(p == 2)
    def _p2():
        @pl.when(k == 0)
        def _recip():
            rg_s[...] = jnp.broadcast_to(
                jnp.float32(1.0)
                / jnp.sum(sga[...], axis=1, keepdims=True), (8, 128))
            ra_s[...] = jnp.broadcast_to(
                jnp.float32(1.0)
                / jnp.sum(saa[...], axis=1, keepdims=True), (8, 128))

        rg = rg_s[...][:, :1]
        ra = ra_s[...][:, :1]
        y = jnp.maximum(g_s[:, cols] * rg, eps)
        y_ref[...] = y
        y2_ref[...] = y
        ma = ma_s[...][:, :1]
        ss_ref[...] = jnp.exp(a_ref[...] - ma) * ra


def kernel(alpha):
    y, y2, ss = pl.pallas_call(
        _kernel,
        grid=(3, _NC),
        in_specs=[pl.BlockSpec((8, _W), lambda p, k: (0, k))],
        out_specs=[
            pl.BlockSpec((8, _W), lambda p, k: (0, (p == 2) * k)),
            pl.BlockSpec((8, _W), lambda p, k: (0, (p == 2) * k)),
            pl.BlockSpec((8, _W), lambda p, k: (0, (p == 2) * k)),
        ],
        out_shape=[
            jax.ShapeDtypeStruct((_B, _V), jnp.float32),
            jax.ShapeDtypeStruct((_B, _V), jnp.float32),
            jax.ShapeDtypeStruct((_B, _V), jnp.float32),
        ],
        scratch_shapes=[
            pltpu.VMEM((8, _NC * _W), jnp.float32),
            pltpu.VMEM((8, _W), jnp.float32),
            pltpu.VMEM((8, _W), jnp.float32),
            pltpu.VMEM((8, _W), jnp.float32),
            pltpu.VMEM((8, _W), jnp.float32),
            pltpu.VMEM((8, 128), jnp.float32),
            pltpu.VMEM((8, 128), jnp.float32),
            pltpu.VMEM((8, 128), jnp.float32),
            pltpu.VMEM((8, 128), jnp.float32),
        ],
    )(alpha)
    return (y, y2, ss)
